# Initial kernel scaffold; baseline (speedup 1.0000x reference)
import jax
import jax.numpy as jnp
from jax.experimental import pallas as pl


def kernel(x, wq, wk, wv, ln_g, ln_b, W1, b1, W2, b2):
    raise NotImplementedError("write your pallas kernel here")


# speedup vs baseline: 5.8064x; 5.8064x over previous
import jax
import jax.numpy as jnp
from jax import lax
from jax.experimental import pallas as pl
from jax.experimental.pallas import tpu as pltpu

_NC = 6
_D = 64
_ROWS = 207 * 12
_N = 8
_DM = _NC * _D
_HI = lax.Precision.HIGHEST


def _mean_body(x_ref, o_ref):
    o_ref[0, 0, :] = jnp.sum(x_ref[0], axis=0) * jnp.float32(1.0 / _ROWS)


def _proj_body(xm_ref, wq_ref, wk_ref, q_ref, kf_ref):
    xm = xm_ref[...]
    q_ref[...] = jnp.dot(xm, wq_ref[...], precision=_HI)
    kf_ref[...] = jnp.dot(xm, wk_ref[...], precision=_HI)


def _sim_body(q_ref, k2_ref, s_ref, r_ref, lg_ref):
    for bi in range(_N):
        qb = q_ref[_NC * bi:_NC * (bi + 1), :]
        k2b = k2_ref[_DM * bi:_DM * (bi + 1), :]
        b = lax.dot_general(qb, k2b, (((1,), (1,)), ((), ())),
                            precision=_HI)
        fs, rcols = [], []
        for j in range(_NC):
            seg = b[:, _D * j:_D * (j + 1)]
            r_j = jnp.sqrt(jnp.sum(seg * seg, axis=1, keepdims=True))
            fs.append(seg / r_j)
            rcols.append(r_j)
        scols = []
        for j1 in range(_NC):
            for j2 in range(_NC):
                scols.append(
                    (jnp.sum(fs[j1] * fs[j2], axis=1, keepdims=True) + 1.0) * 0.5)
        s_ref[bi] = jnp.concatenate(scols, axis=1)
        r_ref[bi] = jnp.concatenate(rcols, axis=1)
        ktm = jnp.sum(k2b, axis=1, keepdims=True) * jnp.float32(1.0 / _D)
        lcols = [jnp.dot(qb, ktm[_D * j:_D * (j + 1), :], precision=_HI)
                 for j in range(_NC)]
        lg_ref[bi] = jnp.concatenate(lcols, axis=1)


def _greedy_body(s_ref, r_ref, lg_ref, wt_ref):
    R = _N * _NC
    S = s_ref[...]
    rs = r_ref[...]
    lg = lg_ref[...]
    i6 = lax.broadcasted_iota(jnp.int32, (R, _NC), 1)
    ni = lax.broadcasted_iota(jnp.int32, (R, _NC), 0) % _NC
    ni1 = ni[:, 0:1]
    one = jnp.float32(1.0)
    zero = jnp.float32(0.0)
    nimask = jnp.where(i6 == ni, zero, one)
    lz = []
    for j in range(_NC):
        row = S[:, _NC * j:_NC * (j + 1)] * nimask
        lz.append(jnp.where(ni1 == j, zero, row))
    diag = jnp.concatenate([S[:, 7 * i:7 * i + 1] for i in range(_NC)], axis=1)
    diag = jnp.where(i6 == ni, jnp.float32(-1e20), diag)

    def argmax_first(score):
        mx = jnp.max(score, axis=1, keepdims=True)
        cand = jnp.where(score == mx, i6, 99)
        jmin = jnp.min(cand, axis=1, keepdims=True)
        return jnp.where(i6 == jmin, one, zero), mx

    ohj, _ = argmax_first(diag * rs)
    selected = ohj
    cm = [jnp.zeros((R, _NC), jnp.float32) for _ in range(_NC)]
    stopped = jnp.zeros((R, 1), jnp.float32)
    for it in range(1, _NC):
        cjk = [jnp.sum(cm[k] * ohj, axis=1, keepdims=True) for k in range(_NC)]
        dots = cjk[0] * cm[0]
        for k in range(1, _NC):
            dots = dots + cjk[k] * cm[k]
        l_j = ohj[:, 0:1] * lz[0]
        for j in range(1, _NC):
            l_j = l_j + ohj[:, j:j + 1] * lz[j]
        diag_j = jnp.sum(diag * ohj, axis=1, keepdims=True)
        e = (l_j - dots) / jnp.sqrt(diag_j)
        zf = (one - selected) * nimask
        e = jnp.where(zf > 0.5, e, zero)
        d2 = diag - e * e
        d2 = jnp.where(ohj > 0.5, jnp.float32(-1e20), d2)
        ohjn, mx = argmax_first(d2 * rs)
        brk = jnp.where(mx < one, one, zero)
        upd = stopped < 0.5
        cm[it] = jnp.where(upd, e, cm[it])
        diag = jnp.where(upd, d2, diag)
        ohj = jnp.where(upd, ohjn, ohj)
        selected = jnp.maximum(selected,
                               ohjn * (one - brk) * (one - stopped))
        stopped = jnp.maximum(stopped, brk)
    selb = selected > 0.5
    ml = jnp.max(jnp.where(selb, lg, jnp.float32(-1e30)), axis=1,
                 keepdims=True)
    ex = jnp.where(selb, jnp.exp(lg - ml), zero)
    wt_ref[...] = ex / jnp.sum(ex, axis=1, keepdims=True)


def _fused_body(x_ref, wt_ref, wv_ref, g_ref, b_ref, w1_ref, b1_ref,
                w2_ref, b2_ref, o_ref):
    xb = x_ref[0]
    wt = wt_ref[0]
    segs = [xb[:, _D * c:_D * (c + 1)] for c in range(_NC)]
    g = g_ref[...]
    bb = b_ref[...]

    def ln(t):
        mu = jnp.mean(t, axis=1, keepdims=True)
        ctr = t - mu
        var = jnp.mean(ctr * ctr, axis=1, keepdims=True)
        return ctr * lax.rsqrt(var + 1e-5) * g + bb

    wv = wv_ref[...]
    ys = []
    for c in range(_NC):
        acc = segs[0] * wt[c:c + 1, 0:1]
        for cp in range(1, _NC):
            acc = acc + segs[cp] * wt[c:c + 1, cp:cp + 1]
        ys.append(ln(jnp.dot(acc, wv, precision=_HI) + segs[c]))
    outs = []
    for p in range(_NC // 2):
        pair = jnp.concatenate([ys[2 * p], ys[2 * p + 1]], axis=1)
        u = jnp.dot(pair, w1_ref[...], precision=_HI) + b1_ref[...]
        h = 0.5 * u * (1.0 + lax.erf(u * jnp.float32(0.7071067811865476)))
        z = jnp.dot(h, w2_ref[...], precision=_HI) + b2_ref[...]
        t = pair + z
        outs.append(ln(t[:, :_D]))
        outs.append(ln(t[:, _D:]))
    o_ref[0] = jnp.concatenate(outs, axis=1)


def kernel(x, wq, wk, wv, ln_g, ln_b, W1, b1, W2, b2):
    n = x.shape[0]
    x3 = x.reshape(n, _ROWS, _DM)

    xm = pl.pallas_call(
        _mean_body,
        grid=(n,),
        in_specs=[pl.BlockSpec((1, _ROWS, _DM), lambda i: (i, 0, 0))],
        out_specs=pl.BlockSpec((1, 1, _DM), lambda i: (i, 0, 0)),
        out_shape=jax.ShapeDtypeStruct((n, 1, _DM), jnp.float32),
    )(x3)
    xmr = xm.reshape(n * _NC, _D)

    q, kflat = pl.pallas_call(
        _proj_body,
        out_shape=(jax.ShapeDtypeStruct((n * _NC, _D), jnp.float32),
                   jax.ShapeDtypeStruct((n * _NC, _D * _D), jnp.float32)),
    )(xmr, wq, wk)
    k2 = kflat.reshape(n * _DM, _D)

    S, rr, lg = pl.pallas_call(
        _sim_body,
        out_shape=(jax.ShapeDtypeStruct((n, _NC, _NC * _NC), jnp.float32),
                   jax.ShapeDtypeStruct((n, _NC, _NC), jnp.float32),
                   jax.ShapeDtypeStruct((n, _NC, _NC), jnp.float32)),
    )(q, k2)

    wt = pl.pallas_call(
        _greedy_body,
        out_shape=jax.ShapeDtypeStruct((n * _NC, _NC), jnp.float32),
    )(S.reshape(n * _NC, _NC * _NC), rr.reshape(n * _NC, _NC),
      lg.reshape(n * _NC, _NC))
    wt3 = wt.reshape(n, _NC, _NC)

    bdW1 = jax.scipy.linalg.block_diag(W1, W1)
    bdW2 = jax.scipy.linalg.block_diag(W2, W2)
    b1t = jnp.tile(b1, 2).reshape(1, 2 * _D)
    b2t = jnp.tile(b2, 2).reshape(1, 2 * _D)

    out = pl.pallas_call(
        _fused_body,
        grid=(n,),
        in_specs=[
            pl.BlockSpec((1, _ROWS, _DM), lambda i: (i, 0, 0)),
            pl.BlockSpec((1, _NC, _NC), lambda i: (i, 0, 0)),
            pl.BlockSpec((_D, _D), lambda i: (0, 0)),
            pl.BlockSpec((1, _D), lambda i: (0, 0)),
            pl.BlockSpec((1, _D), lambda i: (0, 0)),
            pl.BlockSpec((2 * _D, 2 * _D), lambda i: (0, 0)),
            pl.BlockSpec((1, 2 * _D), lambda i: (0, 0)),
            pl.BlockSpec((2 * _D, 2 * _D), lambda i: (0, 0)),
            pl.BlockSpec((1, 2 * _D), lambda i: (0, 0)),
        ],
        out_specs=pl.BlockSpec((1, _ROWS, _DM), lambda i: (i, 0, 0)),
        out_shape=jax.ShapeDtypeStruct((n, _ROWS, _DM), jnp.float32),
    )(x3, wt3, wv, ln_g.reshape(1, _D), ln_b.reshape(1, _D),
      bdW1, b1t, bdW2, b2t)

    return out.reshape(x.shape)
